# baseline (speedup 1.0000x reference)
import functools

import jax
import jax.numpy as jnp
from jax import lax
from jax.experimental import pallas as pl
from jax.experimental.pallas import tpu as pltpu
from jax.experimental.pallas import tpu_sc as plsc

N = 10000
E = 320000
F = 128
O = 16

NC = 2
NS = 16
NW = NC * NS
CH = 250
NCH = 40
EPW = E // NW
NP = 10240
RPT = NP // NS

def _sc_mesh():
    return plsc.VectorSubcoreMesh(core_axis_name="c", subcore_axis_name="s")


def _vrsqrt(x):
    magic = jnp.asarray(0x5F3759DF, jnp.int32)
    i = plsc.bitcast(x, jnp.int32)
    y = plsc.bitcast(magic - lax.shift_right_logical(i, 1), jnp.float32)
    for _ in range(3):
        y = y * (1.5 - 0.5 * x * y * y)
    return y


def _ring(table, sidx_v, didx_v, rows_v, agg, gsem, ssem):
    def gath(j, b):
        pltpu.async_copy(table.at[sidx_v.at[j]], rows_v.at[b], gsem[b])

    def wait_g(j, b):
        pltpu.make_async_copy(table.at[sidx_v.at[j]], rows_v.at[b],
                              gsem[b]).wait()

    def scat(j, b):
        pltpu.async_copy(rows_v.at[b], agg.at[didx_v.at[j]], ssem[b], add=True)

    def wait_s(j, b):
        pltpu.make_async_copy(rows_v.at[b], agg.at[didx_v.at[j]],
                              ssem[b]).wait()

    gath(0, 0)
    gath(1, 1)
    gath(2, 2)
    gath(3, 3)
    wait_g(0, 0)
    scat(0, 0)

    def body(kk, carry):
        j0 = 1 + 4 * kk
        for m in range(4):
            j = j0 + m
            wait_s(j - 1, m)
            gath(j + 3, m)
            wait_g(j, (1 + m) % 4)
            scat(j, (1 + m) % 4)
        return carry

    lax.fori_loop(0, (NCH - 4) // 4, body, 0)
    for j in range(NCH - 3, NCH):
        wait_g(j, j % 4)
        scat(j, j % 4)
    for j in range(NCH - 4, NCH):
        wait_s(j, j % 4)


def _make_deg():
    @functools.partial(
        pl.kernel,
        out_type=pltpu.HBM((NC, NP // O, O), jnp.float32),
        mesh=_sc_mesh(),
        scratch_types=[
            pltpu.VMEM((NCH, CH), jnp.int32),
            pltpu.VMEM((CH, O), jnp.float32),
            pltpu.VMEM((RPT, O), jnp.float32),
            pltpu.VMEM((RPT // O, O), jnp.float32),
            pltpu.VMEM_SHARED((NP, O), jnp.float32),
        ],
        compiler_params=pltpu.CompilerParams(use_tc_tiling_on_sc=False,
                                             needs_layout_passes=False),
    )
    def deg(didx, ones, zrows, out, didx_v, ones_v, zbuf_v, cbuf_v, agg):
        c = lax.axis_index("c")
        s = lax.axis_index("s")
        w = s * NC + c
        pltpu.sync_copy(didx.at[w], didx_v)
        pltpu.sync_copy(ones, ones_v)
        pltpu.sync_copy(zrows, zbuf_v)
        pltpu.sync_copy(zbuf_v, agg.at[pl.ds(s * RPT, RPT)])
        plsc.subcore_barrier()

        def body(j, carry):
            pltpu.sync_copy(ones_v, agg.at[didx_v.at[j]], add=True)
            return carry

        lax.fori_loop(0, NCH, body, 0)
        plsc.subcore_barrier()
        pltpu.sync_copy(agg.at[pl.ds(s * RPT, RPT)], zbuf_v)
        lanes = jnp.arange(O, dtype=jnp.int32)

        def compact(g, carry):
            vals = plsc.load_gather(zbuf_v, [g * O + lanes, lanes * 0])
            cbuf_v[g, :] = vals
            return carry

        lax.fori_loop(0, RPT // O, compact, 0)
        pltpu.sync_copy(cbuf_v, out.at[c, pl.ds(s * (RPT // O), RPT // O)])

    return deg


def _make_prop1():
    W2 = 2 * O

    @functools.partial(
        pl.kernel,
        out_type=(
            pltpu.HBM((NC, NP, W2), jnp.float32),
            pltpu.HBM((NP // O, O), jnp.float32),
        ),
        mesh=_sc_mesh(),
        scratch_types=[
            pltpu.VMEM((NCH, CH), jnp.int32),
            pltpu.VMEM((NCH, CH), jnp.int32),
            pltpu.VMEM((4, CH, W2), jnp.float32),
            pltpu.VMEM((RPT, W2), jnp.float32),
            pltpu.VMEM((2, RPT // O, O), jnp.float32),
            pltpu.VMEM((RPT // O, O), jnp.float32),
            pltpu.VMEM_SHARED((NP, W2), jnp.float32),
            pltpu.VMEM_SHARED((NP, W2), jnp.float32),
            [pltpu.SemaphoreType.DMA] * 4,
            [pltpu.SemaphoreType.DMA] * 4,
        ],
        compiler_params=pltpu.CompilerParams(use_tc_tiling_on_sc=False,
                                             needs_layout_passes=False),
    )
    def prop1(uvr, degp, sidx, didx, zrows, p1, dinvs,
              sidx_v, didx_v, rows_v, ubuf, dbuf, dinv_v, agg, tbl, gsem, ssem):
        c = lax.axis_index("c")
        s = lax.axis_index("s")
        w = s * NC + c
        r0 = s * RPT
        pltpu.sync_copy(sidx.at[w], sidx_v)
        pltpu.sync_copy(didx.at[w], didx_v)
        g0 = s * (RPT // O)
        pltpu.sync_copy(degp.at[0, pl.ds(g0, RPT // O)], dbuf.at[0])
        pltpu.sync_copy(degp.at[1, pl.ds(g0, RPT // O)], dbuf.at[1])

        @pl.when(s < NS - 1)
        def _():
            pltpu.sync_copy(uvr.at[pl.ds(s * RPT, RPT)], ubuf)

        @pl.when(s == NS - 1)
        def _():
            pltpu.sync_copy(uvr.at[pl.ds((NS - 1) * RPT, N - (NS - 1) * RPT)],
                            ubuf.at[pl.ds(0, N - (NS - 1) * RPT)])

        def dinv_build(g, carry):
            deg = jnp.maximum(dbuf[0, g, :] + dbuf[1, g, :], 1.0)
            dinv_v[g, :] = _vrsqrt(deg)
            return carry

        lax.fori_loop(0, RPT // O, dinv_build, 0)

        def build(g, carry):
            yv = dinv_v[g, :]
            for k in range(O):
                y = yv[k]
                i = g * O + k
                ubuf[i, pl.ds(0, O)] = ubuf[i, pl.ds(0, O)] * y
                ubuf[i, pl.ds(O, O)] = ubuf[i, pl.ds(O, O)] * y
            return carry

        lax.fori_loop(0, RPT // O, build, 0)
        pltpu.sync_copy(ubuf, tbl.at[pl.ds(r0, RPT)])

        @pl.when(c == 0)
        def _():
            pltpu.sync_copy(dinv_v, dinvs.at[pl.ds(g0, RPT // O)])
        pltpu.sync_copy(zrows, ubuf)
        pltpu.sync_copy(ubuf, agg.at[pl.ds(r0, RPT)])
        plsc.subcore_barrier()
        _ring(tbl, sidx_v, didx_v, rows_v, agg, gsem, ssem)
        plsc.subcore_barrier()
        pltpu.sync_copy(agg.at[pl.ds(r0, RPT)], ubuf)
        pltpu.sync_copy(ubuf, p1.at[c, pl.ds(r0, RPT)])

    return prop1


def _make_prop2():
    @functools.partial(
        pl.kernel,
        out_type=pltpu.HBM((NC, NP, O), jnp.float32),
        mesh=_sc_mesh(),
        scratch_types=[
            pltpu.VMEM((NCH, CH), jnp.int32),
            pltpu.VMEM((NCH, CH), jnp.int32),
            pltpu.VMEM((4, CH, O), jnp.float32),
            pltpu.VMEM((2, RPT, 2 * O), jnp.float32),
            pltpu.VMEM((RPT // O, O), jnp.float32),
            pltpu.VMEM((RPT, O), jnp.float32),
            pltpu.VMEM((RPT, O), jnp.float32),
            pltpu.VMEM_SHARED((NP, O), jnp.float32),
            pltpu.VMEM_SHARED((NP, O), jnp.float32),
            [pltpu.SemaphoreType.DMA] * 4,
            [pltpu.SemaphoreType.DMA] * 4,
        ],
        compiler_params=pltpu.CompilerParams(use_tc_tiling_on_sc=False,
                                             needs_layout_passes=False),
    )
    def prop2(p1, dinvs, sidx, didx, zrows, r,
              sidx_v, didx_v, rows_v, pbuf, dinv_v, ht_v, wt_v, agg, wtbl,
              gsem, ssem):
        c = lax.axis_index("c")
        s = lax.axis_index("s")
        w = s * NC + c
        r0 = s * RPT
        pltpu.sync_copy(sidx.at[w], sidx_v)
        pltpu.sync_copy(didx.at[w], didx_v)
        pltpu.sync_copy(p1.at[0, pl.ds(r0, RPT)], pbuf.at[0])
        pltpu.sync_copy(p1.at[1, pl.ds(r0, RPT)], pbuf.at[1])
        g0 = s * (RPT // O)
        pltpu.sync_copy(dinvs.at[pl.ds(g0, RPT // O)], dinv_v)

        def build(g, carry):
            yv = dinv_v[g, :]
            for k in range(O):
                y = yv[k]
                i = g * O + k
                pu = pbuf[0, i, pl.ds(0, O)] + pbuf[1, i, pl.ds(0, O)]
                pv = pbuf[0, i, pl.ds(O, O)] + pbuf[1, i, pl.ds(O, O)]
                ht_v[i, :] = -(y * pu)
                wt_v[i, :] = (y * y) * pv
            return carry

        lax.fori_loop(0, RPT // O, build, 0)
        pltpu.sync_copy(wt_v, wtbl.at[pl.ds(r0, RPT)])
        pltpu.sync_copy(zrows, wt_v)
        pltpu.sync_copy(wt_v, agg.at[pl.ds(r0, RPT)])
        plsc.subcore_barrier()
        _ring(wtbl, sidx_v, didx_v, rows_v, agg, gsem, ssem)
        plsc.subcore_barrier()
        pltpu.sync_copy(agg.at[pl.ds(r0, RPT)], wt_v)

        own_half = (s // (NS // NC)) == c

        @pl.when(own_half)
        def _():
            def fin(g, carry):
                yv = dinv_v[g, :]
                for k in range(O):
                    i = g * O + k
                    wt_v[i, :] = wt_v[i, :] * yv[k] + ht_v[i, :]
                return carry
            lax.fori_loop(0, RPT // O, fin, 0)

        @pl.when(jnp.logical_not(own_half))
        def _():
            def fin(g, carry):
                yv = dinv_v[g, :]
                for k in range(O):
                    i = g * O + k
                    wt_v[i, :] = wt_v[i, :] * yv[k]
                return carry
            lax.fori_loop(0, RPT // O, fin, 0)

        pltpu.sync_copy(wt_v, r.at[c, pl.ds(r0, RPT)])

    return prop2


def _tc_mm(x, w0, b0, w1, b1):
    def body(x_ref, w0_ref, b0_ref, w1_ref, b1_ref, uv_ref, c_ref):
        xb = x_ref[...]
        w0b = w0_ref[...]
        w1b = w1_ref[...]
        a0 = jnp.dot(xb, w0b[:F], preferred_element_type=jnp.float32) + b0_ref[...]
        bb0 = jnp.dot(xb, w0b[F:], preferred_element_type=jnp.float32)
        w1a = w1b[:F]
        w1bb = w1b[F:]
        c_ref[...] = jnp.dot(a0, w1a, preferred_element_type=jnp.float32) + b1_ref[...]
        u = (jnp.dot(bb0, w1a, preferred_element_type=jnp.float32)
             + jnp.dot(a0, w1bb, preferred_element_type=jnp.float32))
        v = jnp.dot(bb0, w1bb, preferred_element_type=jnp.float32)
        uv_ref[...] = jnp.concatenate([u, v], axis=1)

    return pl.pallas_call(
        body,
        out_shape=[
            jax.ShapeDtypeStruct((N, 2 * O), jnp.float32),
            jax.ShapeDtypeStruct((N, O), jnp.float32),
        ],
    )(x, w0, b0, w1, b1)


def _tc_final(cc, rr):
    def body(c_ref, r_ref, o_ref):
        ra = r_ref[...]
        o_ref[...] = c_ref[...] + ra[0, :N] + ra[1, :N]

    return pl.pallas_call(
        body,
        out_shape=jax.ShapeDtypeStruct((N, O), jnp.float32),
    )(cc, rr)


_deg = _make_deg()
_prop1 = _make_prop1()
_prop2 = _make_prop2()


def kernel(features, edge_index, W0, b0, W1, b1):
    src_r = edge_index[0].reshape(NW, NCH, CH)
    dst_r = edge_index[1].reshape(NW, NCH, CH)
    zr16 = jnp.zeros((RPT, O), jnp.float32)
    zr32 = jnp.zeros((RPT, 2 * O), jnp.float32)
    ones = jnp.ones((CH, O), jnp.float32)
    b0r = b0.reshape(1, F)
    b1r = b1.reshape(1, O)

    degp = _deg(dst_r, ones, zr16)
    uvr, cc = _tc_mm(features, W0, b0r, W1, b1r)
    p1, dinvs = _prop1(uvr, degp, src_r, dst_r, zr32)
    rr = _prop2(p1, dinvs, src_r, dst_r, zr16)
    return _tc_final(cc, rr)


# speedup vs baseline: 1.0113x; 1.0113x over previous
import functools

import jax
import jax.numpy as jnp
from jax import lax
from jax.experimental import pallas as pl
from jax.experimental.pallas import tpu as pltpu
from jax.experimental.pallas import tpu_sc as plsc

N = 10000
E = 320000
F = 128
O = 16

NC = 2
NS = 16
NW = NC * NS
CH = 320
NCH = 32
EP = NW * NCH * CH
EPW = EP // NW
NP = 10240
RPT = NP // NS

def _sc_mesh():
    return plsc.VectorSubcoreMesh(core_axis_name="c", subcore_axis_name="s")


def _vrsqrt(x):
    magic = jnp.asarray(0x5F3759DF, jnp.int32)
    i = plsc.bitcast(x, jnp.int32)
    y = plsc.bitcast(magic - lax.shift_right_logical(i, 1), jnp.float32)
    for _ in range(3):
        y = y * (1.5 - 0.5 * x * y * y)
    return y


def _ring(table, sidx_v, didx_v, rows_v, agg, gsem, ssem):
    def gath(j, b):
        pltpu.async_copy(table.at[sidx_v.at[j]], rows_v.at[b], gsem[b])

    def wait_g(j, b):
        pltpu.make_async_copy(table.at[sidx_v.at[j]], rows_v.at[b],
                              gsem[b]).wait()

    def scat(j, b):
        pltpu.async_copy(rows_v.at[b], agg.at[didx_v.at[j]], ssem[b], add=True)

    def wait_s(j, b):
        pltpu.make_async_copy(rows_v.at[b], agg.at[didx_v.at[j]],
                              ssem[b]).wait()

    gath(0, 0)
    gath(1, 1)
    gath(2, 2)
    gath(3, 3)
    wait_g(0, 0)
    scat(0, 0)

    def body(kk, carry):
        j0 = 1 + 4 * kk
        for m in range(4):
            j = j0 + m
            wait_s(j - 1, m)
            gath(j + 3, m)
            wait_g(j, (1 + m) % 4)
            scat(j, (1 + m) % 4)
        return carry

    lax.fori_loop(0, (NCH - 4) // 4, body, 0)
    for j in range(NCH - 3, NCH):
        wait_g(j, j % 4)
        scat(j, j % 4)
    for j in range(NCH - 4, NCH):
        wait_s(j, j % 4)


def _make_deg():
    @functools.partial(
        pl.kernel,
        out_type=pltpu.HBM((NC, NP // O, O), jnp.float32),
        mesh=_sc_mesh(),
        scratch_types=[
            pltpu.VMEM((NCH, CH), jnp.int32),
            pltpu.VMEM((CH, O), jnp.float32),
            pltpu.VMEM((RPT, O), jnp.float32),
            pltpu.VMEM((RPT // O, O), jnp.float32),
            pltpu.VMEM_SHARED((NP, O), jnp.float32),
        ],
        compiler_params=pltpu.CompilerParams(use_tc_tiling_on_sc=False,
                                             needs_layout_passes=False),
    )
    def deg(didx, ones, zrows, out, didx_v, ones_v, zbuf_v, cbuf_v, agg):
        c = lax.axis_index("c")
        s = lax.axis_index("s")
        w = s * NC + c
        pltpu.sync_copy(didx.at[w], didx_v)
        pltpu.sync_copy(ones, ones_v)
        pltpu.sync_copy(zrows, zbuf_v)
        pltpu.sync_copy(zbuf_v, agg.at[pl.ds(s * RPT, RPT)])
        plsc.subcore_barrier()

        def body(j, carry):
            pltpu.sync_copy(ones_v, agg.at[didx_v.at[j]], add=True)
            return carry

        lax.fori_loop(0, NCH, body, 0)
        plsc.subcore_barrier()
        pltpu.sync_copy(agg.at[pl.ds(s * RPT, RPT)], zbuf_v)
        lanes = jnp.arange(O, dtype=jnp.int32)

        def compact(g, carry):
            vals = plsc.load_gather(zbuf_v, [g * O + lanes, lanes * 0])
            cbuf_v[g, :] = vals
            return carry

        lax.fori_loop(0, RPT // O, compact, 0)
        pltpu.sync_copy(cbuf_v, out.at[c, pl.ds(s * (RPT // O), RPT // O)])

    return deg


def _make_prop1():
    W2 = 2 * O

    @functools.partial(
        pl.kernel,
        out_type=(
            pltpu.HBM((NC, NP, W2), jnp.float32),
            pltpu.HBM((NP // O, O), jnp.float32),
        ),
        mesh=_sc_mesh(),
        scratch_types=[
            pltpu.VMEM((NCH, CH), jnp.int32),
            pltpu.VMEM((NCH, CH), jnp.int32),
            pltpu.VMEM((4, CH, W2), jnp.float32),
            pltpu.VMEM((RPT, W2), jnp.float32),
            pltpu.VMEM((2, RPT // O, O), jnp.float32),
            pltpu.VMEM((RPT // O, O), jnp.float32),
            pltpu.VMEM_SHARED((NP, W2), jnp.float32),
            pltpu.VMEM_SHARED((NP, W2), jnp.float32),
            [pltpu.SemaphoreType.DMA] * 4,
            [pltpu.SemaphoreType.DMA] * 4,
        ],
        compiler_params=pltpu.CompilerParams(use_tc_tiling_on_sc=False,
                                             needs_layout_passes=False),
    )
    def prop1(uvr, degp, sidx, didx, zrows, p1, dinvs,
              sidx_v, didx_v, rows_v, ubuf, dbuf, dinv_v, agg, tbl, gsem, ssem):
        c = lax.axis_index("c")
        s = lax.axis_index("s")
        w = s * NC + c
        r0 = s * RPT
        pltpu.sync_copy(sidx.at[w], sidx_v)
        pltpu.sync_copy(didx.at[w], didx_v)
        g0 = s * (RPT // O)
        pltpu.sync_copy(degp.at[0, pl.ds(g0, RPT // O)], dbuf.at[0])
        pltpu.sync_copy(degp.at[1, pl.ds(g0, RPT // O)], dbuf.at[1])

        @pl.when(s < NS - 1)
        def _():
            pltpu.sync_copy(uvr.at[pl.ds(s * RPT, RPT)], ubuf)

        @pl.when(s == NS - 1)
        def _():
            pltpu.sync_copy(uvr.at[pl.ds((NS - 1) * RPT, N - (NS - 1) * RPT)],
                            ubuf.at[pl.ds(0, N - (NS - 1) * RPT)])

        def dinv_build(g, carry):
            deg = jnp.maximum(dbuf[0, g, :] + dbuf[1, g, :], 1.0)
            dinv_v[g, :] = _vrsqrt(deg)
            return carry

        lax.fori_loop(0, RPT // O, dinv_build, 0)

        def build(g, carry):
            yv = dinv_v[g, :]
            for k in range(O):
                y = yv[k]
                i = g * O + k
                ubuf[i, pl.ds(0, O)] = ubuf[i, pl.ds(0, O)] * y
                ubuf[i, pl.ds(O, O)] = ubuf[i, pl.ds(O, O)] * y
            return carry

        lax.fori_loop(0, RPT // O, build, 0)
        pltpu.sync_copy(ubuf, tbl.at[pl.ds(r0, RPT)])

        @pl.when(c == 0)
        def _():
            pltpu.sync_copy(dinv_v, dinvs.at[pl.ds(g0, RPT // O)])
        pltpu.sync_copy(zrows, ubuf)
        pltpu.sync_copy(ubuf, agg.at[pl.ds(r0, RPT)])
        plsc.subcore_barrier()
        _ring(tbl, sidx_v, didx_v, rows_v, agg, gsem, ssem)
        plsc.subcore_barrier()
        pltpu.sync_copy(agg.at[pl.ds(r0, RPT)], ubuf)
        pltpu.sync_copy(ubuf, p1.at[c, pl.ds(r0, RPT)])

    return prop1


def _make_prop2():
    @functools.partial(
        pl.kernel,
        out_type=pltpu.HBM((NC, NP, O), jnp.float32),
        mesh=_sc_mesh(),
        scratch_types=[
            pltpu.VMEM((NCH, CH), jnp.int32),
            pltpu.VMEM((NCH, CH), jnp.int32),
            pltpu.VMEM((4, CH, O), jnp.float32),
            pltpu.VMEM((2, RPT, 2 * O), jnp.float32),
            pltpu.VMEM((RPT // O, O), jnp.float32),
            pltpu.VMEM((RPT, O), jnp.float32),
            pltpu.VMEM((RPT, O), jnp.float32),
            pltpu.VMEM_SHARED((NP, O), jnp.float32),
            pltpu.VMEM_SHARED((NP, O), jnp.float32),
            [pltpu.SemaphoreType.DMA] * 4,
            [pltpu.SemaphoreType.DMA] * 4,
        ],
        compiler_params=pltpu.CompilerParams(use_tc_tiling_on_sc=False,
                                             needs_layout_passes=False),
    )
    def prop2(p1, dinvs, sidx, didx, zrows, r,
              sidx_v, didx_v, rows_v, pbuf, dinv_v, ht_v, wt_v, agg, wtbl,
              gsem, ssem):
        c = lax.axis_index("c")
        s = lax.axis_index("s")
        w = s * NC + c
        r0 = s * RPT
        pltpu.sync_copy(sidx.at[w], sidx_v)
        pltpu.sync_copy(didx.at[w], didx_v)
        pltpu.sync_copy(p1.at[0, pl.ds(r0, RPT)], pbuf.at[0])
        pltpu.sync_copy(p1.at[1, pl.ds(r0, RPT)], pbuf.at[1])
        g0 = s * (RPT // O)
        pltpu.sync_copy(dinvs.at[pl.ds(g0, RPT // O)], dinv_v)

        def build(g, carry):
            yv = dinv_v[g, :]
            for k in range(O):
                y = yv[k]
                i = g * O + k
                pu = pbuf[0, i, pl.ds(0, O)] + pbuf[1, i, pl.ds(0, O)]
                pv = pbuf[0, i, pl.ds(O, O)] + pbuf[1, i, pl.ds(O, O)]
                ht_v[i, :] = -(y * pu)
                wt_v[i, :] = (y * y) * pv
            return carry

        lax.fori_loop(0, RPT // O, build, 0)
        pltpu.sync_copy(wt_v, wtbl.at[pl.ds(r0, RPT)])
        pltpu.sync_copy(zrows, wt_v)
        pltpu.sync_copy(wt_v, agg.at[pl.ds(r0, RPT)])
        plsc.subcore_barrier()
        _ring(wtbl, sidx_v, didx_v, rows_v, agg, gsem, ssem)
        plsc.subcore_barrier()
        pltpu.sync_copy(agg.at[pl.ds(r0, RPT)], wt_v)

        own_half = (s // (NS // NC)) == c

        @pl.when(own_half)
        def _():
            def fin(g, carry):
                yv = dinv_v[g, :]
                for k in range(O):
                    i = g * O + k
                    wt_v[i, :] = wt_v[i, :] * yv[k] + ht_v[i, :]
                return carry
            lax.fori_loop(0, RPT // O, fin, 0)

        @pl.when(jnp.logical_not(own_half))
        def _():
            def fin(g, carry):
                yv = dinv_v[g, :]
                for k in range(O):
                    i = g * O + k
                    wt_v[i, :] = wt_v[i, :] * yv[k]
                return carry
            lax.fori_loop(0, RPT // O, fin, 0)

        pltpu.sync_copy(wt_v, r.at[c, pl.ds(r0, RPT)])

    return prop2


def _tc_mm(x, w0, b0, w1, b1):
    def body(x_ref, w0_ref, b0_ref, w1_ref, b1_ref, uv_ref, c_ref):
        xb = x_ref[...]
        w0b = w0_ref[...]
        w1b = w1_ref[...]
        a0 = jnp.dot(xb, w0b[:F], preferred_element_type=jnp.float32) + b0_ref[...]
        bb0 = jnp.dot(xb, w0b[F:], preferred_element_type=jnp.float32)
        w1a = w1b[:F]
        w1bb = w1b[F:]
        c_ref[...] = jnp.dot(a0, w1a, preferred_element_type=jnp.float32) + b1_ref[...]
        u = (jnp.dot(bb0, w1a, preferred_element_type=jnp.float32)
             + jnp.dot(a0, w1bb, preferred_element_type=jnp.float32))
        v = jnp.dot(bb0, w1bb, preferred_element_type=jnp.float32)
        uv_ref[...] = jnp.concatenate([u, v], axis=1)

    return pl.pallas_call(
        body,
        out_shape=[
            jax.ShapeDtypeStruct((N, 2 * O), jnp.float32),
            jax.ShapeDtypeStruct((N, O), jnp.float32),
        ],
    )(x, w0, b0, w1, b1)


def _tc_final(cc, rr):
    def body(c_ref, r_ref, o_ref):
        ra = r_ref[...]
        o_ref[...] = c_ref[...] + ra[0, :N] + ra[1, :N]

    return pl.pallas_call(
        body,
        out_shape=jax.ShapeDtypeStruct((N, O), jnp.float32),
    )(cc, rr)


_deg = _make_deg()
_prop1 = _make_prop1()
_prop2 = _make_prop2()


def kernel(features, edge_index, W0, b0, W1, b1):
    npad = EP - E
    pad_src = (jnp.arange(npad, dtype=jnp.int32) * 131) % N
    pad_dst = N + (jnp.arange(npad, dtype=jnp.int32) % (NP - N))
    src_r = jnp.concatenate([edge_index[0], pad_src]).reshape(NW, NCH, CH)
    dst_r = jnp.concatenate([edge_index[1], pad_dst]).reshape(NW, NCH, CH)
    zr16 = jnp.zeros((RPT, O), jnp.float32)
    zr32 = jnp.zeros((RPT, 2 * O), jnp.float32)
    ones = jnp.ones((CH, O), jnp.float32)
    b0r = b0.reshape(1, F)
    b1r = b1.reshape(1, O)

    degp = _deg(dst_r, ones, zr16)
    uvr, cc = _tc_mm(features, W0, b0r, W1, b1r)
    p1, dinvs = _prop1(uvr, degp, src_r, dst_r, zr32)
    rr = _prop2(p1, dinvs, src_r, dst_r, zr16)
    return _tc_final(cc, rr)


# speedup vs baseline: 1.0410x; 1.0294x over previous
import functools

import jax
import jax.numpy as jnp
from jax import lax
from jax.experimental import pallas as pl
from jax.experimental.pallas import tpu as pltpu
from jax.experimental.pallas import tpu_sc as plsc

N = 10000
E = 320000
F = 128
O = 16

NC = 2
NS = 16
NW = NC * NS
CH = 320
NCH = 32
EP = NW * NCH * CH
EPW = EP // NW
NP = 10240
RPT = NP // NS

def _sc_mesh():
    return plsc.VectorSubcoreMesh(core_axis_name="c", subcore_axis_name="s")


def _vrsqrt(x):
    magic = jnp.asarray(0x5F3759DF, jnp.int32)
    i = plsc.bitcast(x, jnp.int32)
    y = plsc.bitcast(magic - lax.shift_right_logical(i, 1), jnp.float32)
    for _ in range(3):
        y = y * (1.5 - 0.5 * x * y * y)
    return y


def _ring(table, sidx_v, didx_v, rows_v, agg, gsem, ssem):
    def gath(j, b):
        pltpu.async_copy(table.at[sidx_v.at[j]], rows_v.at[b], gsem[b])

    def wait_g(j, b):
        pltpu.make_async_copy(table.at[sidx_v.at[j]], rows_v.at[b],
                              gsem[b]).wait()

    def scat(j, b):
        pltpu.async_copy(rows_v.at[b], agg.at[didx_v.at[j]], ssem[b], add=True)

    def wait_s(j, b):
        pltpu.make_async_copy(rows_v.at[b], agg.at[didx_v.at[j]],
                              ssem[b]).wait()

    gath(0, 0)
    gath(1, 1)
    gath(2, 2)
    gath(3, 3)
    wait_g(0, 0)
    scat(0, 0)

    def body(kk, carry):
        j0 = 1 + 4 * kk
        for m in range(4):
            j = j0 + m
            wait_s(j - 1, m)
            gath(j + 3, m)
            wait_g(j, (1 + m) % 4)
            scat(j, (1 + m) % 4)
        return carry

    lax.fori_loop(0, (NCH - 4) // 4, body, 0)
    for j in range(NCH - 3, NCH):
        wait_g(j, j % 4)
        scat(j, j % 4)
    for j in range(NCH - 4, NCH):
        wait_s(j, j % 4)


def _make_deg():
    @functools.partial(
        pl.kernel,
        out_type=pltpu.HBM((NC, NP // O, O), jnp.float32),
        mesh=_sc_mesh(),
        scratch_types=[
            pltpu.VMEM((NCH, CH), jnp.int32),
            pltpu.VMEM((CH, O), jnp.float32),
            pltpu.VMEM((RPT, O), jnp.float32),
            pltpu.VMEM((RPT // O, O), jnp.float32),
            pltpu.VMEM_SHARED((NP, O), jnp.float32),
        ],
        compiler_params=pltpu.CompilerParams(use_tc_tiling_on_sc=False,
                                             needs_layout_passes=False),
    )
    def deg(didx, ones, zrows, out, didx_v, ones_v, zbuf_v, cbuf_v, agg):
        c = lax.axis_index("c")
        s = lax.axis_index("s")
        w = s * NC + c
        pltpu.sync_copy(didx.at[w], didx_v)
        pltpu.sync_copy(ones, ones_v)
        pltpu.sync_copy(zrows, zbuf_v)
        pltpu.sync_copy(zbuf_v, agg.at[pl.ds(s * RPT, RPT)])
        plsc.subcore_barrier()

        def body(j, carry):
            pltpu.sync_copy(ones_v, agg.at[didx_v.at[j]], add=True)
            return carry

        lax.fori_loop(0, NCH, body, 0)
        plsc.subcore_barrier()
        pltpu.sync_copy(agg.at[pl.ds(s * RPT, RPT)], zbuf_v)
        lanes = jnp.arange(O, dtype=jnp.int32)

        def compact(g, carry):
            vals = plsc.load_gather(zbuf_v, [g * O + lanes, lanes * 0])
            cbuf_v[g, :] = vals
            return carry

        lax.fori_loop(0, RPT // O, compact, 0)
        pltpu.sync_copy(cbuf_v, out.at[c, pl.ds(s * (RPT // O), RPT // O)])

    return deg


def _make_prop1():
    W2 = 2 * O

    @functools.partial(
        pl.kernel,
        out_type=(
            pltpu.HBM((NC, NP, W2), jnp.float32),
            pltpu.HBM((NP // O, O), jnp.float32),
        ),
        mesh=_sc_mesh(),
        scratch_types=[
            pltpu.VMEM((NCH, CH), jnp.int32),
            pltpu.VMEM((NCH, CH), jnp.int32),
            pltpu.VMEM((4, CH, W2), jnp.float32),
            pltpu.VMEM((RPT, W2), jnp.float32),
            pltpu.VMEM((2, RPT // O, O), jnp.float32),
            pltpu.VMEM((RPT // O, O), jnp.float32),
            pltpu.VMEM_SHARED((NP, W2), jnp.float32),
            pltpu.VMEM_SHARED((NP, W2), jnp.float32),
            [pltpu.SemaphoreType.DMA] * 4,
            [pltpu.SemaphoreType.DMA] * 4,
        ],
        compiler_params=pltpu.CompilerParams(use_tc_tiling_on_sc=False,
                                             needs_layout_passes=False),
    )
    def prop1(uvr, degp, sidx, didx, zrows, p1, dinvs,
              sidx_v, didx_v, rows_v, ubuf, dbuf, dinv_v, agg, tbl, gsem, ssem):
        c = lax.axis_index("c")
        s = lax.axis_index("s")
        w = s * NC + c
        r0 = s * RPT
        g0 = s * (RPT // O)
        pltpu.async_copy(sidx.at[w], sidx_v, gsem[0])
        pltpu.async_copy(didx.at[w], didx_v, gsem[1])
        pltpu.async_copy(degp.at[0, pl.ds(g0, RPT // O)], dbuf.at[0], gsem[2])
        pltpu.async_copy(degp.at[1, pl.ds(g0, RPT // O)], dbuf.at[1], gsem[3])

        @pl.when(s < NS - 1)
        def _():
            pltpu.async_copy(uvr.at[pl.ds(s * RPT, RPT)], ubuf, ssem[0])
            pltpu.make_async_copy(uvr.at[pl.ds(s * RPT, RPT)], ubuf,
                                  ssem[0]).wait()

        @pl.when(s == NS - 1)
        def _():
            pltpu.async_copy(uvr.at[pl.ds((NS - 1) * RPT, N - (NS - 1) * RPT)],
                             ubuf.at[pl.ds(0, N - (NS - 1) * RPT)], ssem[0])
            pltpu.make_async_copy(
                uvr.at[pl.ds((NS - 1) * RPT, N - (NS - 1) * RPT)],
                ubuf.at[pl.ds(0, N - (NS - 1) * RPT)], ssem[0]).wait()

        pltpu.make_async_copy(sidx.at[w], sidx_v, gsem[0]).wait()
        pltpu.make_async_copy(didx.at[w], didx_v, gsem[1]).wait()
        pltpu.make_async_copy(degp.at[0, pl.ds(g0, RPT // O)], dbuf.at[0],
                              gsem[2]).wait()
        pltpu.make_async_copy(degp.at[1, pl.ds(g0, RPT // O)], dbuf.at[1],
                              gsem[3]).wait()

        def dinv_build(g, carry):
            deg = jnp.maximum(dbuf[0, g, :] + dbuf[1, g, :], 1.0)
            dinv_v[g, :] = _vrsqrt(deg)
            return carry

        lax.fori_loop(0, RPT // O, dinv_build, 0)

        def build(g, carry):
            yv = dinv_v[g, :]
            for k in range(O):
                y = yv[k]
                i = g * O + k
                ubuf[i, pl.ds(0, O)] = ubuf[i, pl.ds(0, O)] * y
                ubuf[i, pl.ds(O, O)] = ubuf[i, pl.ds(O, O)] * y
            return carry

        lax.fori_loop(0, RPT // O, build, 0)
        pltpu.sync_copy(ubuf, tbl.at[pl.ds(r0, RPT)])

        @pl.when(c == 0)
        def _():
            pltpu.sync_copy(dinv_v, dinvs.at[pl.ds(g0, RPT // O)])
        pltpu.sync_copy(zrows, ubuf)
        pltpu.sync_copy(ubuf, agg.at[pl.ds(r0, RPT)])
        plsc.subcore_barrier()
        _ring(tbl, sidx_v, didx_v, rows_v, agg, gsem, ssem)
        plsc.subcore_barrier()
        pltpu.sync_copy(agg.at[pl.ds(r0, RPT)], ubuf)
        pltpu.sync_copy(ubuf, p1.at[c, pl.ds(r0, RPT)])

    return prop1


def _make_prop2():
    @functools.partial(
        pl.kernel,
        out_type=pltpu.HBM((NC, NP, O), jnp.float32),
        mesh=_sc_mesh(),
        scratch_types=[
            pltpu.VMEM((NCH, CH), jnp.int32),
            pltpu.VMEM((NCH, CH), jnp.int32),
            pltpu.VMEM((4, CH, O), jnp.float32),
            pltpu.VMEM((2, RPT, 2 * O), jnp.float32),
            pltpu.VMEM((RPT // O, O), jnp.float32),
            pltpu.VMEM((RPT, O), jnp.float32),
            pltpu.VMEM((RPT, O), jnp.float32),
            pltpu.VMEM_SHARED((NP, O), jnp.float32),
            pltpu.VMEM_SHARED((NP, O), jnp.float32),
            [pltpu.SemaphoreType.DMA] * 4,
            [pltpu.SemaphoreType.DMA] * 4,
        ],
        compiler_params=pltpu.CompilerParams(use_tc_tiling_on_sc=False,
                                             needs_layout_passes=False),
    )
    def prop2(p1, dinvs, sidx, didx, zrows, r,
              sidx_v, didx_v, rows_v, pbuf, dinv_v, ht_v, wt_v, agg, wtbl,
              gsem, ssem):
        c = lax.axis_index("c")
        s = lax.axis_index("s")
        w = s * NC + c
        r0 = s * RPT
        g0 = s * (RPT // O)
        pltpu.async_copy(sidx.at[w], sidx_v, gsem[0])
        pltpu.async_copy(didx.at[w], didx_v, gsem[1])
        pltpu.async_copy(p1.at[0, pl.ds(r0, RPT)], pbuf.at[0], gsem[2])
        pltpu.async_copy(p1.at[1, pl.ds(r0, RPT)], pbuf.at[1], gsem[3])
        pltpu.async_copy(dinvs.at[pl.ds(g0, RPT // O)], dinv_v, ssem[0])
        pltpu.make_async_copy(sidx.at[w], sidx_v, gsem[0]).wait()
        pltpu.make_async_copy(didx.at[w], didx_v, gsem[1]).wait()
        pltpu.make_async_copy(p1.at[0, pl.ds(r0, RPT)], pbuf.at[0],
                              gsem[2]).wait()
        pltpu.make_async_copy(p1.at[1, pl.ds(r0, RPT)], pbuf.at[1],
                              gsem[3]).wait()
        pltpu.make_async_copy(dinvs.at[pl.ds(g0, RPT // O)], dinv_v,
                              ssem[0]).wait()

        def build(g, carry):
            yv = dinv_v[g, :]
            for k in range(O):
                y = yv[k]
                i = g * O + k
                pu = pbuf[0, i, pl.ds(0, O)] + pbuf[1, i, pl.ds(0, O)]
                pv = pbuf[0, i, pl.ds(O, O)] + pbuf[1, i, pl.ds(O, O)]
                ht_v[i, :] = -(y * pu)
                wt_v[i, :] = (y * y) * pv
            return carry

        lax.fori_loop(0, RPT // O, build, 0)
        pltpu.sync_copy(wt_v, wtbl.at[pl.ds(r0, RPT)])
        pltpu.sync_copy(zrows, wt_v)
        pltpu.sync_copy(wt_v, agg.at[pl.ds(r0, RPT)])
        plsc.subcore_barrier()
        _ring(wtbl, sidx_v, didx_v, rows_v, agg, gsem, ssem)
        plsc.subcore_barrier()
        pltpu.sync_copy(agg.at[pl.ds(r0, RPT)], wt_v)

        own_half = (s // (NS // NC)) == c

        @pl.when(own_half)
        def _():
            def fin(g, carry):
                yv = dinv_v[g, :]
                for k in range(O):
                    i = g * O + k
                    wt_v[i, :] = wt_v[i, :] * yv[k] + ht_v[i, :]
                return carry
            lax.fori_loop(0, RPT // O, fin, 0)

        @pl.when(jnp.logical_not(own_half))
        def _():
            def fin(g, carry):
                yv = dinv_v[g, :]
                for k in range(O):
                    i = g * O + k
                    wt_v[i, :] = wt_v[i, :] * yv[k]
                return carry
            lax.fori_loop(0, RPT // O, fin, 0)

        pltpu.sync_copy(wt_v, r.at[c, pl.ds(r0, RPT)])

    return prop2


def _tc_mm(x, w0, b0, w1, b1):
    def body(x_ref, w0_ref, b0_ref, w1_ref, b1_ref, uv_ref, c_ref):
        xb = x_ref[...].astype(jnp.bfloat16)
        w0b = w0_ref[...].astype(jnp.bfloat16)
        w1b = w1_ref[...]
        a0 = jnp.dot(xb, w0b[:F], preferred_element_type=jnp.float32) + b0_ref[...]
        bb0 = jnp.dot(xb, w0b[F:], preferred_element_type=jnp.float32)
        w1a = w1b[:F]
        w1bb = w1b[F:]
        c_ref[...] = jnp.dot(a0, w1a, preferred_element_type=jnp.float32) + b1_ref[...]
        u = (jnp.dot(bb0, w1a, preferred_element_type=jnp.float32)
             + jnp.dot(a0, w1bb, preferred_element_type=jnp.float32))
        v = jnp.dot(bb0, w1bb, preferred_element_type=jnp.float32)
        uv_ref[...] = jnp.concatenate([u, v], axis=1)

    return pl.pallas_call(
        body,
        out_shape=[
            jax.ShapeDtypeStruct((N, 2 * O), jnp.float32),
            jax.ShapeDtypeStruct((N, O), jnp.float32),
        ],
    )(x, w0, b0, w1, b1)


def _tc_final(cc, rr):
    def body(c_ref, r_ref, o_ref):
        ra = r_ref[...]
        o_ref[...] = c_ref[...] + ra[0, :N] + ra[1, :N]

    return pl.pallas_call(
        body,
        out_shape=jax.ShapeDtypeStruct((N, O), jnp.float32),
    )(cc, rr)


_deg = _make_deg()
_prop1 = _make_prop1()
_prop2 = _make_prop2()


def kernel(features, edge_index, W0, b0, W1, b1):
    npad = EP - E
    pad_src = (jnp.arange(npad, dtype=jnp.int32) * 131) % N
    pad_dst = N + (jnp.arange(npad, dtype=jnp.int32) % (NP - N))
    src_r = jnp.concatenate([edge_index[0], pad_src]).reshape(NW, NCH, CH)
    dst_r = jnp.concatenate([edge_index[1], pad_dst]).reshape(NW, NCH, CH)
    zr16 = jnp.zeros((RPT, O), jnp.float32)
    zr32 = jnp.zeros((RPT, 2 * O), jnp.float32)
    ones = jnp.ones((CH, O), jnp.float32)
    b0r = b0.reshape(1, F)
    b1r = b1.reshape(1, O)

    degp = _deg(dst_r, ones, zr16)
    uvr, cc = _tc_mm(features, W0, b0r, W1, b1r)
    p1, dinvs = _prop1(uvr, degp, src_r, dst_r, zr32)
    rr = _prop2(p1, dinvs, src_r, dst_r, zr16)
    return _tc_final(cc, rr)
